# baseline (speedup 1.0000x reference)
import functools

import jax
import jax.numpy as jnp
from jax import lax
from jax.experimental import pallas as pl
from jax.experimental.pallas import tpu as pltpu
from jax.experimental.pallas import tpu_sc as plsc

N_NODES = 10000
NPAD = 10240
N_EDGES = 320000
D_IN = 128
D_H1 = 128
D_H2 = 64

NC = 2
NS = 16
NW = NC * NS
CHUNK = 128
NCHUNK = 160
E_PAD = NS * NCHUNK * CHUNK
ZONE = NPAD // NS

_mesh = lambda: plsc.VectorSubcoreMesh(core_axis_name="c", subcore_axis_name="s")


def _zero_rows(rows, nrow, d):
    zero_pat = jnp.zeros((16,), jnp.float32)

    def zstep(i, carry):
        for j in range(d // 16):
            rows[i, pl.ds(j * 16, 16)] = zero_pat
        return carry

    lax.fori_loop(0, nrow, zstep, 0)


def _zero_zone(rows, acc_sh, s):
    for i in range(ZONE // CHUNK):
        pltpu.sync_copy(rows, acc_sh.at[pl.ds(s * ZONE + i * CHUNK, CHUNK)])


def _zone_out(acc_sh, rows, out_hbm, c, s):
    for i in range(ZONE // CHUNK):
        off = s * ZONE + i * CHUNK
        pltpu.sync_copy(acc_sh.at[pl.ds(off, CHUNK)], rows)
        pltpu.sync_copy(rows, out_hbm.at[c, pl.ds(off, CHUNK)])


HW = 8


def _hist_body(dst_hbm, ones_hbm, zeros_hbm, out_hbm, dstv, obuf, zbuf, acc_sh, sem):
    c = lax.axis_index("c")
    s = lax.axis_index("s")
    idx_cp = pltpu.make_async_copy(dst_hbm.at[s, pl.ds(c * (NCHUNK // 2), NCHUNK // 2)], dstv, sem)
    idx_cp.start()
    pltpu.sync_copy(ones_hbm, obuf)
    pltpu.sync_copy(zeros_hbm, zbuf)
    _zero_zone(zbuf, acc_sh, s)
    idx_cp.wait()
    plsc.subcore_barrier()

    def step(k, carry):
        pltpu.async_copy(obuf, acc_sh.at[dstv.at[k]], sem, add=True)
        return carry

    lax.fori_loop(0, NCHUNK // 2, step, 0)

    def drain(k, carry):
        pltpu.make_async_copy(obuf, acc_sh.at[dstv.at[k]], sem).wait()
        return carry

    lax.fori_loop(0, NCHUNK // 2, drain, 0)
    plsc.subcore_barrier()
    _zone_out(acc_sh, zbuf, out_hbm, c, s)


def _make_hist():
    return pl.kernel(
        _hist_body,
        out_type=jax.ShapeDtypeStruct((NC, NPAD, HW), jnp.float32),
        mesh=_mesh(),
        compiler_params=pltpu.CompilerParams(use_tc_tiling_on_sc=False),
        scratch_types=[
            pltpu.VMEM((NCHUNK // 2, CHUNK), jnp.int32),
            pltpu.VMEM((CHUNK, HW), jnp.float32),
            pltpu.VMEM((CHUNK, HW), jnp.float32),
            pltpu.VMEM_SHARED((NPAD, HW), jnp.float32),
            pltpu.SemaphoreType.DMA,
        ],
    )


def _agg_body(d, ga_hbm, gb_hbm, src_hbm, dst_hbm, out_hbm,
              srcv, dstv, rows0, rows1, acc_sh, isem, gsem0, gsem1):
    c = lax.axis_index("c")
    s = lax.axis_index("s")
    src_cp = pltpu.make_async_copy(src_hbm.at[s], srcv, isem)
    dst_cp = pltpu.make_async_copy(dst_hbm.at[s], dstv, isem)
    src_cp.start()
    dst_cp.start()
    _zero_rows(rows0, CHUNK, d)
    _zero_zone(rows0, acc_sh, s)
    src_cp.wait()
    dst_cp.wait()
    plsc.subcore_barrier()

    bufs = (rows0, rows1)
    sems = (gsem0, gsem1)

    def gath(k, buf, sem):
        @pl.when(c == 0)
        def _():
            pltpu.async_copy(ga_hbm.at[srcv.at[k]], buf, sem)

        @pl.when(c == 1)
        def _():
            pltpu.async_copy(gb_hbm.at[srcv.at[k]], buf, sem)

    gath(0, rows0, gsem0)

    def pair(p, carry):
        for b in range(2):
            k = 2 * p + b
            @pl.when(k < NCHUNK - 1)
            def _():
                gath(k + 1, bufs[1 - b], sems[1 - b])
            pltpu.make_async_copy(ga_hbm.at[srcv.at[k]], bufs[b], sems[b]).wait()
            pltpu.sync_copy(bufs[b], acc_sh.at[dstv.at[k]], add=True)
        return carry

    lax.fori_loop(0, NCHUNK // 2, pair, 0)
    plsc.subcore_barrier()
    for i in range(ZONE // CHUNK):
        off = s * ZONE + i * CHUNK
        pltpu.sync_copy(acc_sh.at[pl.ds(off, CHUNK)], rows1)
        pltpu.sync_copy(rows1, out_hbm.at[c, pl.ds(off, CHUNK)])


def _make_agg(d):
    return pl.kernel(
        functools.partial(_agg_body, d),
        out_type=jax.ShapeDtypeStruct((NC, NPAD, d), jnp.float32),
        mesh=_mesh(),
        compiler_params=pltpu.CompilerParams(use_tc_tiling_on_sc=False),
        scratch_types=[
            pltpu.VMEM((NCHUNK, CHUNK), jnp.int32),
            pltpu.VMEM((NCHUNK, CHUNK), jnp.int32),
            pltpu.VMEM((CHUNK, d), jnp.float32),
            pltpu.VMEM((CHUNK, d), jnp.float32),
            pltpu.VMEM_SHARED((NPAD, d), jnp.float32),
            pltpu.SemaphoreType.DMA,
            pltpu.SemaphoreType.DMA,
            pltpu.SemaphoreType.DMA,
        ],
    )


DAGG = D_H1 // 2
DH2H = D_H2 // 2
BLK = 512
GRID = NPAD // BLK


def _tc_a_body(x_ref, wa_ref, wb_ref, h0_ref, h1_ref, ga_ref, gb_ref, dis_ref):
    deg = 1.0 + h0_ref[:, 0:1] + h1_ref[:, 0:1]
    dis = lax.rsqrt(deg)
    ga_ref[...] = dis * jnp.dot(x_ref[...], wa_ref[...], preferred_element_type=jnp.float32)
    gb_ref[...] = dis * jnp.dot(x_ref[...], wb_ref[...], preferred_element_type=jnp.float32)
    dis_ref[...] = dis


def _tc_b_body(aa_ref, ab_ref, ga_ref, gb_ref, dis_ref,
               b_ref, wa_ref, wb_ref, oa_ref, ob_ref):
    dis = dis_ref[...]
    o1a = dis * (aa_ref[...] + ga_ref[...])
    o1b = dis * (ab_ref[...] + gb_ref[...])
    o1 = jnp.maximum(jnp.concatenate([o1a, o1b], axis=1) + b_ref[...], 0.0)
    oa_ref[...] = dis * jnp.dot(o1, wa_ref[...], preferred_element_type=jnp.float32)
    ob_ref[...] = dis * jnp.dot(o1, wb_ref[...], preferred_element_type=jnp.float32)


def _tc_c_body(aa_ref, ab_ref, ga_ref, gb_ref, dis_ref, b_ref, out_ref):
    dis = dis_ref[...]
    o = jnp.concatenate([aa_ref[...] + ga_ref[...], ab_ref[...] + gb_ref[...]], axis=1)
    out_ref[...] = jnp.maximum(dis * o + b_ref[...], 0.0)


def _row_spec(d):
    return pl.BlockSpec((BLK, d), lambda i: (i, 0))


def _full_spec(r, c):
    return pl.BlockSpec((r, c), lambda i: (0, 0))


_tc_a = pl.pallas_call(
    _tc_a_body,
    grid=(GRID,),
    in_specs=[_row_spec(D_IN), _full_spec(D_IN, DAGG), _full_spec(D_IN, DAGG),
              _row_spec(HW), _row_spec(HW)],
    out_specs=[_row_spec(DAGG), _row_spec(DAGG), _row_spec(1)],
    out_shape=[jax.ShapeDtypeStruct((NPAD, DAGG), jnp.float32),
               jax.ShapeDtypeStruct((NPAD, DAGG), jnp.float32),
               jax.ShapeDtypeStruct((NPAD, 1), jnp.float32)],
)

_tc_b = pl.pallas_call(
    _tc_b_body,
    grid=(GRID,),
    in_specs=[_row_spec(DAGG)] * 4 + [_row_spec(1),
              _full_spec(1, D_H1), _full_spec(D_H1, DH2H), _full_spec(D_H1, DH2H)],
    out_specs=[_row_spec(DH2H), _row_spec(DH2H)],
    out_shape=[jax.ShapeDtypeStruct((NPAD, DH2H), jnp.float32),
               jax.ShapeDtypeStruct((NPAD, DH2H), jnp.float32)],
)

_tc_c = pl.pallas_call(
    _tc_c_body,
    grid=(GRID,),
    in_specs=[_row_spec(DH2H), _row_spec(DH2H), _row_spec(DH2H), _row_spec(DH2H),
              _row_spec(1), _full_spec(1, D_H2)],
    out_specs=_row_spec(D_H2),
    out_shape=jax.ShapeDtypeStruct((NPAD, D_H2), jnp.float32),
)

_hist = _make_hist()
_agg1 = _make_agg(DAGG)
_agg2 = _make_agg(DH2H)


def kernel(x, edge_index, W1, b1, W2, b2):
    ei = edge_index.astype(jnp.int32)
    pad = N_NODES + jnp.arange(E_PAD - N_EDGES, dtype=jnp.int32) % (NPAD - N_NODES)
    src = jnp.concatenate([ei[0], pad]).reshape(NS, NCHUNK, CHUNK)
    dst = jnp.concatenate([ei[1], pad]).reshape(NS, NCHUNK, CHUNK)
    x_pad = jnp.pad(x, ((0, NPAD - N_NODES), (0, 0)))

    ones8 = jnp.tile(jnp.eye(1, HW, dtype=jnp.float32), (CHUNK, 1))
    zeros8 = jnp.zeros((CHUNK, HW), jnp.float32)
    hist = _hist(dst, ones8, zeros8)
    g1a, g1b, dis = _tc_a(x_pad, W1[:, :DAGG], W1[:, DAGG:], hist[0], hist[1])
    acc1 = _agg1(g1a, g1b, src, dst)
    g2a, g2b = _tc_b(acc1[0], acc1[1], g1a, g1b, dis, b1.reshape(1, -1),
                     W2[:, :DH2H], W2[:, DH2H:])
    acc2 = _agg2(g2a, g2b, src, dst)
    out = _tc_c(acc2[0], acc2[1], g2a, g2b, dis, b2.reshape(1, -1))
    return out[:N_NODES]


# speedup vs baseline: 1.0628x; 1.0628x over previous
import functools

import jax
import jax.numpy as jnp
from jax import lax
from jax.experimental import pallas as pl
from jax.experimental.pallas import tpu as pltpu
from jax.experimental.pallas import tpu_sc as plsc

N_NODES = 10000
NPAD = 10240
N_EDGES = 320000
D_IN = 128
D_H1 = 128
D_H2 = 64

NC = 2
NS = 16
NW = NC * NS
CHUNK = 128
NCHUNK = 160
E_PAD = NS * NCHUNK * CHUNK
ZONE = NPAD // NS

_mesh = lambda: plsc.VectorSubcoreMesh(core_axis_name="c", subcore_axis_name="s")


def _zero_rows(rows, nrow, d):
    zero_pat = jnp.zeros((16,), jnp.float32)

    def zstep(i, carry):
        for j in range(d // 16):
            rows[i, pl.ds(j * 16, 16)] = zero_pat
        return carry

    lax.fori_loop(0, nrow, zstep, 0)


def _zero_zone(rows, acc_sh, s):
    for i in range(ZONE // CHUNK):
        pltpu.sync_copy(rows, acc_sh.at[pl.ds(s * ZONE + i * CHUNK, CHUNK)])


def _zone_out(acc_sh, rows, out_hbm, c, s):
    for i in range(ZONE // CHUNK):
        off = s * ZONE + i * CHUNK
        pltpu.sync_copy(acc_sh.at[pl.ds(off, CHUNK)], rows)
        pltpu.sync_copy(rows, out_hbm.at[c, pl.ds(off, CHUNK)])


HW = 8


def _hist_body(dst_hbm, ones_hbm, zeros_hbm, out_hbm, dstv, obuf, zbuf, acc_sh, sem):
    c = lax.axis_index("c")
    s = lax.axis_index("s")
    idx_cp = pltpu.make_async_copy(dst_hbm.at[s, pl.ds(c * (NCHUNK // 2), NCHUNK // 2)], dstv, sem)
    idx_cp.start()
    pltpu.sync_copy(ones_hbm, obuf)
    pltpu.sync_copy(zeros_hbm, zbuf)
    _zero_zone(zbuf, acc_sh, s)
    idx_cp.wait()
    plsc.subcore_barrier()

    def step(k, carry):
        pltpu.async_copy(obuf, acc_sh.at[dstv.at[k]], sem, add=True)
        return carry

    lax.fori_loop(0, NCHUNK // 2, step, 0)

    def drain(k, carry):
        pltpu.make_async_copy(obuf, acc_sh.at[dstv.at[k]], sem).wait()
        return carry

    lax.fori_loop(0, NCHUNK // 2, drain, 0)
    plsc.subcore_barrier()
    _zone_out(acc_sh, zbuf, out_hbm, c, s)


def _make_hist():
    return pl.kernel(
        _hist_body,
        out_type=jax.ShapeDtypeStruct((NC, NPAD, HW), jnp.float32),
        mesh=_mesh(),
        compiler_params=pltpu.CompilerParams(use_tc_tiling_on_sc=False),
        scratch_types=[
            pltpu.VMEM((NCHUNK // 2, CHUNK), jnp.int32),
            pltpu.VMEM((CHUNK, HW), jnp.float32),
            pltpu.VMEM((CHUNK, HW), jnp.float32),
            pltpu.VMEM_SHARED((NPAD, HW), jnp.float32),
            pltpu.SemaphoreType.DMA,
        ],
    )


def _agg_body(d, ga_hbm, gb_hbm, src_hbm, dst_hbm, out_hbm,
              srcv, dstv, rows0, rows1, acc_sh, isem, gsem0, gsem1):
    c = lax.axis_index("c")
    s = lax.axis_index("s")
    src_cp = pltpu.make_async_copy(src_hbm.at[s], srcv, isem)
    dst_cp = pltpu.make_async_copy(dst_hbm.at[s], dstv, isem)
    src_cp.start()
    dst_cp.start()
    _zero_rows(rows0, CHUNK, d)
    _zero_zone(rows0, acc_sh, s)
    src_cp.wait()
    dst_cp.wait()
    plsc.subcore_barrier()

    bufs = (rows0, rows1)
    sems = (gsem0, gsem1)

    def gath(k, buf, sem):
        @pl.when(c == 0)
        def _():
            pltpu.async_copy(ga_hbm.at[srcv.at[k]], buf, sem)

        @pl.when(c == 1)
        def _():
            pltpu.async_copy(gb_hbm.at[srcv.at[k]], buf, sem)

    gath(0, rows0, gsem0)

    def pair(p, carry):
        for b in range(2):
            k = 2 * p + b
            @pl.when(k < NCHUNK - 1)
            def _():
                gath(k + 1, bufs[1 - b], sems[1 - b])
            pltpu.make_async_copy(ga_hbm.at[srcv.at[k]], bufs[b], sems[b]).wait()
            pltpu.sync_copy(bufs[b], acc_sh.at[dstv.at[k]], add=True)
        return carry

    lax.fori_loop(0, NCHUNK // 2, pair, 0)
    plsc.subcore_barrier()
    for i in range(ZONE // CHUNK):
        off = s * ZONE + i * CHUNK
        pltpu.sync_copy(acc_sh.at[pl.ds(off, CHUNK)], rows1)
        pltpu.sync_copy(rows1, out_hbm.at[c, pl.ds(off, CHUNK)])


def _make_agg(d):
    return pl.kernel(
        functools.partial(_agg_body, d),
        out_type=jax.ShapeDtypeStruct((NC, NPAD, d), jnp.float32),
        mesh=_mesh(),
        compiler_params=pltpu.CompilerParams(use_tc_tiling_on_sc=False),
        scratch_types=[
            pltpu.VMEM((NCHUNK, CHUNK), jnp.int32),
            pltpu.VMEM((NCHUNK, CHUNK), jnp.int32),
            pltpu.VMEM((CHUNK, d), jnp.float32),
            pltpu.VMEM((CHUNK, d), jnp.float32),
            pltpu.VMEM_SHARED((NPAD, d), jnp.float32),
            pltpu.SemaphoreType.DMA,
            pltpu.SemaphoreType.DMA,
            pltpu.SemaphoreType.DMA,
        ],
    )


DAGG = D_H1 // 2
DH2H = D_H2 // 2
BLK = 512
GRID = NPAD // BLK


def _tc_a_body(x_ref, wa_ref, wb_ref, hist_ref, ga_ref, gb_ref, dis_ref):
    deg = 1.0 + hist_ref[0, :, 0:1] + hist_ref[1, :, 0:1]
    dis = lax.rsqrt(deg)
    ga_ref[...] = dis * jnp.dot(x_ref[...], wa_ref[...], preferred_element_type=jnp.float32)
    gb_ref[...] = dis * jnp.dot(x_ref[...], wb_ref[...], preferred_element_type=jnp.float32)
    dis_ref[...] = dis


def _tc_b_body(acc_ref, ga_ref, gb_ref, dis_ref, b_ref, wa_ref, wb_ref,
               oa_ref, ob_ref):
    dis = dis_ref[...]
    o1a = dis * (acc_ref[0] + ga_ref[...])
    o1b = dis * (acc_ref[1] + gb_ref[...])
    o1 = jnp.maximum(jnp.concatenate([o1a, o1b], axis=1) + b_ref[...], 0.0)
    oa_ref[...] = dis * jnp.dot(o1, wa_ref[...], preferred_element_type=jnp.float32)
    ob_ref[...] = dis * jnp.dot(o1, wb_ref[...], preferred_element_type=jnp.float32)


def _tc_c_body(acc_ref, ga_ref, gb_ref, dis_ref, b_ref, out_ref):
    dis = dis_ref[...]
    o = jnp.concatenate([acc_ref[0] + ga_ref[...], acc_ref[1] + gb_ref[...]], axis=1)
    out_ref[...] = jnp.maximum(dis * o + b_ref[...], 0.0)


def _row_spec(d):
    return pl.BlockSpec((BLK, d), lambda i: (i, 0))


def _acc_spec(d):
    return pl.BlockSpec((NC, BLK, d), lambda i: (0, i, 0))


def _full_spec(r, c):
    return pl.BlockSpec((r, c), lambda i: (0, 0))


_tc_a = pl.pallas_call(
    _tc_a_body,
    grid=(GRID,),
    in_specs=[_row_spec(D_IN), _full_spec(D_IN, DAGG), _full_spec(D_IN, DAGG),
              _acc_spec(HW)],
    out_specs=[_row_spec(DAGG), _row_spec(DAGG), _row_spec(1)],
    out_shape=[jax.ShapeDtypeStruct((NPAD, DAGG), jnp.float32),
               jax.ShapeDtypeStruct((NPAD, DAGG), jnp.float32),
               jax.ShapeDtypeStruct((NPAD, 1), jnp.float32)],
)

_tc_b = pl.pallas_call(
    _tc_b_body,
    grid=(GRID,),
    in_specs=[_acc_spec(DAGG), _row_spec(DAGG), _row_spec(DAGG), _row_spec(1),
              _full_spec(1, D_H1), _full_spec(D_H1, DH2H), _full_spec(D_H1, DH2H)],
    out_specs=[_row_spec(DH2H), _row_spec(DH2H)],
    out_shape=[jax.ShapeDtypeStruct((NPAD, DH2H), jnp.float32),
               jax.ShapeDtypeStruct((NPAD, DH2H), jnp.float32)],
)

_tc_c = pl.pallas_call(
    _tc_c_body,
    grid=(GRID,),
    in_specs=[_acc_spec(DH2H), _row_spec(DH2H), _row_spec(DH2H), _row_spec(1),
              _full_spec(1, D_H2)],
    out_specs=_row_spec(D_H2),
    out_shape=jax.ShapeDtypeStruct((N_NODES, D_H2), jnp.float32),
)

_hist = _make_hist()
_agg1 = _make_agg(DAGG)
_agg2 = _make_agg(DH2H)


def kernel(x, edge_index, W1, b1, W2, b2):
    ei = edge_index.astype(jnp.int32)
    pad = N_NODES + jnp.arange(E_PAD - N_EDGES, dtype=jnp.int32) % (NPAD - N_NODES)
    src = jnp.concatenate([ei[0], pad]).reshape(NS, NCHUNK, CHUNK)
    dst = jnp.concatenate([ei[1], pad]).reshape(NS, NCHUNK, CHUNK)
    x_pad = jnp.pad(x, ((0, NPAD - N_NODES), (0, 0)))

    ones8 = jnp.tile(jnp.eye(1, HW, dtype=jnp.float32), (CHUNK, 1))
    zeros8 = jnp.zeros((CHUNK, HW), jnp.float32)
    hist = _hist(dst, ones8, zeros8)
    g1a, g1b, dis = _tc_a(x_pad, W1[:, :DAGG], W1[:, DAGG:], hist)
    acc1 = _agg1(g1a, g1b, src, dst)
    g2a, g2b = _tc_b(acc1, g1a, g1b, dis, b1.reshape(1, -1),
                     W2[:, :DH2H], W2[:, DH2H:])
    acc2 = _agg2(g2a, g2b, src, dst)
    out = _tc_c(acc2, g2a, g2b, dis, b2.reshape(1, -1))
    return out


# speedup vs baseline: 1.1234x; 1.0571x over previous
import functools

import jax
import jax.numpy as jnp
from jax import lax
from jax.experimental import pallas as pl
from jax.experimental.pallas import tpu as pltpu
from jax.experimental.pallas import tpu_sc as plsc

N_NODES = 10000
NPAD = 10240
N_EDGES = 320000
D_IN = 128
D_H1 = 128
D_H2 = 64

NC = 2
NS = 16
NW = NC * NS
CHUNK = 128
NCHUNK = 160
E_PAD = NS * NCHUNK * CHUNK
ZONE = NPAD // NS

_mesh = lambda: plsc.VectorSubcoreMesh(core_axis_name="c", subcore_axis_name="s")


def _zero_rows(rows, nrow, d):
    zero_pat = jnp.zeros((16,), jnp.float32)

    def zstep(i, carry):
        for j in range(d // 16):
            rows[i, pl.ds(j * 16, 16)] = zero_pat
        return carry

    lax.fori_loop(0, nrow, zstep, 0)


def _zero_zone(rows, acc_sh, s):
    for i in range(ZONE // CHUNK):
        pltpu.sync_copy(rows, acc_sh.at[pl.ds(s * ZONE + i * CHUNK, CHUNK)])


def _zone_out(acc_sh, rows, out_hbm, c, s):
    for i in range(ZONE // CHUNK):
        off = s * ZONE + i * CHUNK
        pltpu.sync_copy(acc_sh.at[pl.ds(off, CHUNK)], rows)
        pltpu.sync_copy(rows, out_hbm.at[c, pl.ds(off, CHUNK)])


HW = 8


def _hist_body(dst_hbm, ones_hbm, zeros_hbm, out_hbm, dstv, obuf, zbuf, acc_sh, sem):
    c = lax.axis_index("c")
    s = lax.axis_index("s")
    idx_cp = pltpu.make_async_copy(dst_hbm.at[s, pl.ds(c * (NCHUNK // 2), NCHUNK // 2)], dstv, sem)
    idx_cp.start()
    pltpu.sync_copy(ones_hbm, obuf)
    pltpu.sync_copy(zeros_hbm, zbuf)
    _zero_zone(zbuf, acc_sh, s)
    idx_cp.wait()
    plsc.subcore_barrier()

    def step(k, carry):
        pltpu.async_copy(obuf, acc_sh.at[dstv.at[k]], sem, add=True)
        return carry

    lax.fori_loop(0, NCHUNK // 2, step, 0)

    def drain(k, carry):
        pltpu.make_async_copy(obuf, acc_sh.at[dstv.at[k]], sem).wait()
        return carry

    lax.fori_loop(0, NCHUNK // 2, drain, 0)
    plsc.subcore_barrier()
    _zone_out(acc_sh, zbuf, out_hbm, c, s)


def _make_hist():
    return pl.kernel(
        _hist_body,
        out_type=jax.ShapeDtypeStruct((NC, NPAD, HW), jnp.float32),
        mesh=_mesh(),
        compiler_params=pltpu.CompilerParams(use_tc_tiling_on_sc=False),
        scratch_types=[
            pltpu.VMEM((NCHUNK // 2, CHUNK), jnp.int32),
            pltpu.VMEM((CHUNK, HW), jnp.float32),
            pltpu.VMEM((CHUNK, HW), jnp.float32),
            pltpu.VMEM_SHARED((NPAD, HW), jnp.float32),
            pltpu.SemaphoreType.DMA,
        ],
    )


def _agg_body(d, ga_hbm, gb_hbm, src_hbm, dst_hbm, out_hbm, srcv, dstv,
              rows0, rows1, rows2, rows3, acc_sh, isem,
              g0, g1, g2, g3, s0, s1, s2, s3):
    c = lax.axis_index("c")
    s = lax.axis_index("s")
    src_cp = pltpu.make_async_copy(src_hbm.at[s], srcv, isem)
    dst_cp = pltpu.make_async_copy(dst_hbm.at[s], dstv, isem)
    src_cp.start()
    dst_cp.start()
    _zero_rows(rows0, CHUNK, d)
    _zero_zone(rows0, acc_sh, s)
    src_cp.wait()
    dst_cp.wait()
    plsc.subcore_barrier()

    bufs = (rows0, rows1, rows2, rows3)
    gsems = (g0, g1, g2, g3)
    ssems = (s0, s1, s2, s3)

    def gath(k, b):
        @pl.when(c == 0)
        def _():
            pltpu.async_copy(ga_hbm.at[srcv.at[k]], bufs[b], gsems[b])

        @pl.when(c == 1)
        def _():
            pltpu.async_copy(gb_hbm.at[srcv.at[k]], bufs[b], gsems[b])

    def waitg(k, b):
        pltpu.make_async_copy(ga_hbm.at[srcv.at[k]], bufs[b], gsems[b]).wait()

    def scat(k, b):
        pltpu.async_copy(bufs[b], acc_sh.at[dstv.at[k]], ssems[b], add=True)

    def waits(k, b):
        pltpu.make_async_copy(bufs[b], acc_sh.at[dstv.at[k]], ssems[b]).wait()

    gath(0, 0)
    gath(1, 1)
    for k in range(2):
        waitg(k, k)
        scat(k, k)
        gath(k + 2, k + 2)
    for k in range(2, 4):
        waitg(k, k)
        scat(k, k)
        waits(k - 2, (k + 2) % 4)
        gath(k + 2, (k + 2) % 4)

    def quad(q, carry):
        for b in range(4):
            k = 4 * q + b
            waitg(k, b)
            scat(k, b)
            b2 = (b + 2) % 4

            @pl.when(k + 2 < NCHUNK)
            def _():
                waits(k - 2, b2)
                gath(k + 2, b2)
        return carry

    lax.fori_loop(1, NCHUNK // 4, quad, 0)
    for k in range(NCHUNK - 4, NCHUNK):
        waits(k, k % 4)
    plsc.subcore_barrier()
    for i in range(ZONE // CHUNK):
        off = s * ZONE + i * CHUNK
        pltpu.sync_copy(acc_sh.at[pl.ds(off, CHUNK)], rows1)
        pltpu.sync_copy(rows1, out_hbm.at[c, pl.ds(off, CHUNK)])


def _make_agg(d):
    return pl.kernel(
        functools.partial(_agg_body, d),
        out_type=jax.ShapeDtypeStruct((NC, NPAD, d), jnp.float32),
        mesh=_mesh(),
        compiler_params=pltpu.CompilerParams(use_tc_tiling_on_sc=False),
        scratch_types=[
            pltpu.VMEM((NCHUNK, CHUNK), jnp.int32),
            pltpu.VMEM((NCHUNK, CHUNK), jnp.int32),
            pltpu.VMEM((CHUNK, d), jnp.float32),
            pltpu.VMEM((CHUNK, d), jnp.float32),
            pltpu.VMEM((CHUNK, d), jnp.float32),
            pltpu.VMEM((CHUNK, d), jnp.float32),
            pltpu.VMEM_SHARED((NPAD, d), jnp.float32),
        ] + [pltpu.SemaphoreType.DMA] * 9,
    )


DAGG = D_H1 // 2
DH2H = D_H2 // 2
BLK = 512
GRID = NPAD // BLK


def _tc_a_body(x_ref, wa_ref, wb_ref, hist_ref, ga_ref, gb_ref, dis_ref):
    deg = 1.0 + hist_ref[0, :, 0:1] + hist_ref[1, :, 0:1]
    dis = lax.rsqrt(deg)
    ga_ref[...] = dis * jnp.dot(x_ref[...], wa_ref[...], preferred_element_type=jnp.float32)
    gb_ref[...] = dis * jnp.dot(x_ref[...], wb_ref[...], preferred_element_type=jnp.float32)
    dis_ref[...] = dis


def _tc_b_body(acc_ref, ga_ref, gb_ref, dis_ref, b_ref, wa_ref, wb_ref,
               oa_ref, ob_ref):
    dis = dis_ref[...]
    o1a = dis * (acc_ref[0] + ga_ref[...])
    o1b = dis * (acc_ref[1] + gb_ref[...])
    o1 = jnp.maximum(jnp.concatenate([o1a, o1b], axis=1) + b_ref[...], 0.0)
    oa_ref[...] = dis * jnp.dot(o1, wa_ref[...], preferred_element_type=jnp.float32)
    ob_ref[...] = dis * jnp.dot(o1, wb_ref[...], preferred_element_type=jnp.float32)


def _tc_c_body(acc_ref, ga_ref, gb_ref, dis_ref, b_ref, out_ref):
    dis = dis_ref[...]
    o = jnp.concatenate([acc_ref[0] + ga_ref[...], acc_ref[1] + gb_ref[...]], axis=1)
    out_ref[...] = jnp.maximum(dis * o + b_ref[...], 0.0)


def _row_spec(d):
    return pl.BlockSpec((BLK, d), lambda i: (i, 0))


def _acc_spec(d):
    return pl.BlockSpec((NC, BLK, d), lambda i: (0, i, 0))


def _full_spec(r, c):
    return pl.BlockSpec((r, c), lambda i: (0, 0))


_tc_a = pl.pallas_call(
    _tc_a_body,
    grid=(GRID,),
    in_specs=[_row_spec(D_IN), _full_spec(D_IN, DAGG), _full_spec(D_IN, DAGG),
              _acc_spec(HW)],
    out_specs=[_row_spec(DAGG), _row_spec(DAGG), _row_spec(1)],
    out_shape=[jax.ShapeDtypeStruct((NPAD, DAGG), jnp.float32),
               jax.ShapeDtypeStruct((NPAD, DAGG), jnp.float32),
               jax.ShapeDtypeStruct((NPAD, 1), jnp.float32)],
)

_tc_b = pl.pallas_call(
    _tc_b_body,
    grid=(GRID,),
    in_specs=[_acc_spec(DAGG), _row_spec(DAGG), _row_spec(DAGG), _row_spec(1),
              _full_spec(1, D_H1), _full_spec(D_H1, DH2H), _full_spec(D_H1, DH2H)],
    out_specs=[_row_spec(DH2H), _row_spec(DH2H)],
    out_shape=[jax.ShapeDtypeStruct((NPAD, DH2H), jnp.float32),
               jax.ShapeDtypeStruct((NPAD, DH2H), jnp.float32)],
)

_tc_c = pl.pallas_call(
    _tc_c_body,
    grid=(GRID,),
    in_specs=[_acc_spec(DH2H), _row_spec(DH2H), _row_spec(DH2H), _row_spec(1),
              _full_spec(1, D_H2)],
    out_specs=_row_spec(D_H2),
    out_shape=jax.ShapeDtypeStruct((N_NODES, D_H2), jnp.float32),
)

_hist = _make_hist()
_agg1 = _make_agg(DAGG)
_agg2 = _make_agg(DH2H)


def kernel(x, edge_index, W1, b1, W2, b2):
    ei = edge_index.astype(jnp.int32)
    pad = N_NODES + jnp.arange(E_PAD - N_EDGES, dtype=jnp.int32) % (NPAD - N_NODES)
    src = jnp.concatenate([ei[0], pad]).reshape(NS, NCHUNK, CHUNK)
    dst = jnp.concatenate([ei[1], pad]).reshape(NS, NCHUNK, CHUNK)
    x_pad = jnp.pad(x, ((0, NPAD - N_NODES), (0, 0)))

    ones8 = jnp.tile(jnp.eye(1, HW, dtype=jnp.float32), (CHUNK, 1))
    zeros8 = jnp.zeros((CHUNK, HW), jnp.float32)
    hist = _hist(dst, ones8, zeros8)
    g1a, g1b, dis = _tc_a(x_pad, W1[:, :DAGG], W1[:, DAGG:], hist)
    acc1 = _agg1(g1a, g1b, src, dst)
    g2a, g2b = _tc_b(acc1, g1a, g1b, dis, b1.reshape(1, -1),
                     W2[:, :DH2H], W2[:, DH2H:])
    acc2 = _agg2(g2a, g2b, src, dst)
    out = _tc_c(acc2, g2a, g2b, dis, b2.reshape(1, -1))
    return out


# speedup vs baseline: 1.2605x; 1.1220x over previous
import functools

import jax
import jax.numpy as jnp
from jax import lax
from jax.experimental import pallas as pl
from jax.experimental.pallas import tpu as pltpu
from jax.experimental.pallas import tpu_sc as plsc

N_NODES = 10000
NPAD = 10240
N_EDGES = 320000
D_IN = 128
D_H1 = 128
D_H2 = 64

NC = 2
NS = 16
NW = NC * NS
CHUNK = 128
NCHUNK = 160
E_PAD = NS * NCHUNK * CHUNK
ZONE = NPAD // NS

_mesh = lambda: plsc.VectorSubcoreMesh(core_axis_name="c", subcore_axis_name="s")


def _zero_rows(rows, nrow, d):
    zero_pat = jnp.zeros((16,), jnp.float32)

    def zstep(i, carry):
        for j in range(d // 16):
            rows[i, pl.ds(j * 16, 16)] = zero_pat
        return carry

    lax.fori_loop(0, nrow, zstep, 0)


def _zero_zone(rows, acc_sh, s):
    for i in range(ZONE // CHUNK):
        pltpu.sync_copy(rows, acc_sh.at[pl.ds(s * ZONE + i * CHUNK, CHUNK)])


def _zone_out(acc_sh, rows, out_hbm, c, s):
    for i in range(ZONE // CHUNK):
        off = s * ZONE + i * CHUNK
        pltpu.sync_copy(acc_sh.at[pl.ds(off, CHUNK)], rows)
        pltpu.sync_copy(rows, out_hbm.at[c, pl.ds(off, CHUNK)])


HW = 8


def _hist_body(dst_hbm, ones_hbm, zeros_hbm, out_hbm, dstv, obuf, zbuf, acc_sh, sem):
    c = lax.axis_index("c")
    s = lax.axis_index("s")
    idx_cp = pltpu.make_async_copy(dst_hbm.at[s, pl.ds(c * (NCHUNK // 2), NCHUNK // 2)], dstv, sem)
    idx_cp.start()
    pltpu.sync_copy(ones_hbm, obuf)
    pltpu.sync_copy(zeros_hbm, zbuf)
    _zero_zone(zbuf, acc_sh, s)
    idx_cp.wait()
    plsc.subcore_barrier()

    def step(k, carry):
        pltpu.async_copy(obuf, acc_sh.at[dstv.at[k]], sem, add=True)
        return carry

    lax.fori_loop(0, NCHUNK // 2, step, 0)

    def drain(k, carry):
        pltpu.make_async_copy(obuf, acc_sh.at[dstv.at[k]], sem).wait()
        return carry

    lax.fori_loop(0, NCHUNK // 2, drain, 0)
    plsc.subcore_barrier()
    _zone_out(acc_sh, zbuf, out_hbm, c, s)


def _make_hist():
    return pl.kernel(
        _hist_body,
        out_type=jax.ShapeDtypeStruct((NC, NPAD, HW), jnp.float32),
        mesh=_mesh(),
        compiler_params=pltpu.CompilerParams(use_tc_tiling_on_sc=False),
        scratch_types=[
            pltpu.VMEM((NCHUNK // 2, CHUNK), jnp.int32),
            pltpu.VMEM((CHUNK, HW), jnp.float32),
            pltpu.VMEM((CHUNK, HW), jnp.float32),
            pltpu.VMEM_SHARED((NPAD, HW), jnp.float32),
            pltpu.SemaphoreType.DMA,
        ],
    )


def _agg_body(d, depth, *refs):
    ga_hbm, gb_hbm, src_hbm, dst_hbm, out_hbm, srcv, dstv = refs[:7]
    bufs = refs[7:7 + depth]
    acc_sh = refs[7 + depth]
    isem = refs[8 + depth]
    gsems = refs[9 + depth:9 + 2 * depth]
    ssems = refs[9 + 2 * depth:9 + 3 * depth]
    half = depth // 2
    c = lax.axis_index("c")
    s = lax.axis_index("s")
    src_cp = pltpu.make_async_copy(src_hbm.at[s], srcv, isem)
    dst_cp = pltpu.make_async_copy(dst_hbm.at[s], dstv, isem)
    src_cp.start()
    dst_cp.start()
    _zero_rows(bufs[0], CHUNK, d)
    _zero_zone(bufs[0], acc_sh, s)
    src_cp.wait()
    dst_cp.wait()
    plsc.subcore_barrier()

    def gath(k, b):
        @pl.when(c == 0)
        def _():
            pltpu.async_copy(ga_hbm.at[srcv.at[k]], bufs[b], gsems[b])

        @pl.when(c == 1)
        def _():
            pltpu.async_copy(gb_hbm.at[srcv.at[k]], bufs[b], gsems[b])

    def waitg(k, b):
        pltpu.make_async_copy(ga_hbm.at[srcv.at[k]], bufs[b], gsems[b]).wait()

    def scat(k, b):
        pltpu.async_copy(bufs[b], acc_sh.at[dstv.at[k]], ssems[b], add=True)

    def waits(k, b):
        pltpu.make_async_copy(bufs[b], acc_sh.at[dstv.at[k]], ssems[b]).wait()

    for k in range(half):
        gath(k, k)
    for k in range(half):
        waitg(k, k)
        scat(k, k)
        gath(k + half, k + half)
    for k in range(half, depth):
        waitg(k, k)
        scat(k, k)
        waits(k - half, k - half)
        gath(k + half, k - half)

    def ring(q, carry):
        for b in range(depth):
            k = depth * q + b
            waitg(k, b)
            scat(k, b)
            b2 = (b + half) % depth

            @pl.when(k + half < NCHUNK)
            def _():
                waits(k - half, b2)
                gath(k + half, b2)
        return carry

    lax.fori_loop(1, NCHUNK // depth, ring, 0)
    for k in range(NCHUNK - depth, NCHUNK):
        waits(k, k % depth)
    plsc.subcore_barrier()
    for i in range(ZONE // CHUNK):
        off = s * ZONE + i * CHUNK
        pltpu.sync_copy(acc_sh.at[pl.ds(off, CHUNK)], bufs[1])
        pltpu.sync_copy(bufs[1], out_hbm.at[c, pl.ds(off, CHUNK)])


def _make_agg(d, depth):
    return pl.kernel(
        functools.partial(_agg_body, d, depth),
        out_type=jax.ShapeDtypeStruct((NC, NPAD, d), jnp.float32),
        mesh=_mesh(),
        compiler_params=pltpu.CompilerParams(use_tc_tiling_on_sc=False),
        scratch_types=[
            pltpu.VMEM((NCHUNK, CHUNK), jnp.int32),
            pltpu.VMEM((NCHUNK, CHUNK), jnp.int32),
        ] + [pltpu.VMEM((CHUNK, d), jnp.float32)] * depth + [
            pltpu.VMEM_SHARED((NPAD, d), jnp.float32),
        ] + [pltpu.SemaphoreType.DMA] * (1 + 2 * depth),
    )


DAGG = D_H1 // 2
DH2H = D_H2 // 2
BLK = 1024
GRID = NPAD // BLK


def _tc_a_body(x_ref, wa_ref, wb_ref, hist_ref, ga_ref, gb_ref, dis_ref):
    deg = 1.0 + hist_ref[0, :, 0:1] + hist_ref[1, :, 0:1]
    dis = lax.rsqrt(deg)
    ga_ref[...] = dis * jnp.dot(x_ref[...], wa_ref[...], preferred_element_type=jnp.float32)
    gb_ref[...] = dis * jnp.dot(x_ref[...], wb_ref[...], preferred_element_type=jnp.float32)
    dis_ref[...] = dis


def _tc_b_body(acc_ref, ga_ref, gb_ref, dis_ref, b_ref, wa_ref, wb_ref,
               oa_ref, ob_ref):
    dis = dis_ref[...]
    o1a = dis * (acc_ref[0] + ga_ref[...])
    o1b = dis * (acc_ref[1] + gb_ref[...])
    o1 = jnp.maximum(jnp.concatenate([o1a, o1b], axis=1) + b_ref[...], 0.0)
    oa_ref[...] = dis * jnp.dot(o1, wa_ref[...], preferred_element_type=jnp.float32)
    ob_ref[...] = dis * jnp.dot(o1, wb_ref[...], preferred_element_type=jnp.float32)


def _tc_c_body(acc_ref, ga_ref, gb_ref, dis_ref, b_ref, out_ref):
    dis = dis_ref[...]
    o = jnp.concatenate([acc_ref[0] + ga_ref[...], acc_ref[1] + gb_ref[...]], axis=1)
    out_ref[...] = jnp.maximum(dis * o + b_ref[...], 0.0)


def _row_spec(d):
    return pl.BlockSpec((BLK, d), lambda i: (i, 0))


def _acc_spec(d):
    return pl.BlockSpec((NC, BLK, d), lambda i: (0, i, 0))


def _full_spec(r, c):
    return pl.BlockSpec((r, c), lambda i: (0, 0))


_tc_a = pl.pallas_call(
    _tc_a_body,
    grid=(GRID,),
    in_specs=[_row_spec(D_IN), _full_spec(D_IN, DAGG), _full_spec(D_IN, DAGG),
              _acc_spec(HW)],
    out_specs=[_row_spec(DAGG), _row_spec(DAGG), _row_spec(1)],
    out_shape=[jax.ShapeDtypeStruct((NPAD, DAGG), jnp.float32),
               jax.ShapeDtypeStruct((NPAD, DAGG), jnp.float32),
               jax.ShapeDtypeStruct((NPAD, 1), jnp.float32)],
)

_tc_b = pl.pallas_call(
    _tc_b_body,
    grid=(GRID,),
    in_specs=[_acc_spec(DAGG), _row_spec(DAGG), _row_spec(DAGG), _row_spec(1),
              _full_spec(1, D_H1), _full_spec(D_H1, DH2H), _full_spec(D_H1, DH2H)],
    out_specs=[_row_spec(DH2H), _row_spec(DH2H)],
    out_shape=[jax.ShapeDtypeStruct((NPAD, DH2H), jnp.float32),
               jax.ShapeDtypeStruct((NPAD, DH2H), jnp.float32)],
)

_tc_c = pl.pallas_call(
    _tc_c_body,
    grid=(GRID,),
    in_specs=[_acc_spec(DH2H), _row_spec(DH2H), _row_spec(DH2H), _row_spec(1),
              _full_spec(1, D_H2)],
    out_specs=_row_spec(D_H2),
    out_shape=jax.ShapeDtypeStruct((N_NODES, D_H2), jnp.float32),
)

_hist = _make_hist()
_agg1 = _make_agg(DAGG, 4)
_agg2 = _make_agg(DH2H, 8)


def kernel(x, edge_index, W1, b1, W2, b2):
    ei = edge_index.astype(jnp.int32)
    pad = N_NODES + jnp.arange(E_PAD - N_EDGES, dtype=jnp.int32) % (NPAD - N_NODES)
    src = jnp.concatenate([ei[0], pad]).reshape(NS, NCHUNK, CHUNK)
    dst = jnp.concatenate([ei[1], pad]).reshape(NS, NCHUNK, CHUNK)
    x_pad = jnp.pad(x, ((0, NPAD - N_NODES), (0, 0)))

    ones8 = jnp.tile(jnp.eye(1, HW, dtype=jnp.float32), (CHUNK, 1))
    zeros8 = jnp.zeros((CHUNK, HW), jnp.float32)
    hist = _hist(dst, ones8, zeros8)
    g1a, g1b, dis = _tc_a(x_pad, W1[:, :DAGG], W1[:, DAGG:], hist)
    acc1 = _agg1(g1a, g1b, src, dst)
    g2a, g2b = _tc_b(acc1, g1a, g1b, dis, b1.reshape(1, -1),
                     W2[:, :DH2H], W2[:, DH2H:])
    acc2 = _agg2(g2a, g2b, src, dst)
    out = _tc_c(acc2, g2a, g2b, dis, b2.reshape(1, -1))
    return out
